# baseline (speedup 1.0000x reference)
import functools

import jax
import jax.numpy as jnp
from jax import lax
from jax.experimental import pallas as pl
from jax.experimental.pallas import tpu as pltpu
from jax.experimental.pallas import tpu_sc as plsc

EPS = 1e-8

_N = 1024 * 1024
_C = 16
_L = 16
_NW = 32
_CHUNK = 2048
_NCH = _N // (_NW * _CHUNK)


def _rsqrt16(x):
    i = lax.bitcast_convert_type(x, jnp.int32)
    magic = jnp.full((_L,), 0x5F3759DF, dtype=jnp.int32)
    y = lax.bitcast_convert_type(magic - (i >> 1), jnp.float32)
    for _ in range(3):
        y = y * (1.5 - 0.5 * x * y * y)
    return y


_mesh = plsc.VectorSubcoreMesh(core_axis_name="c", subcore_axis_name="s")


@functools.partial(
    pl.kernel,
    out_type=jax.ShapeDtypeStruct((_NW, _L), jnp.float32),
    mesh=_mesh,
    scratch_types=[
        pltpu.VMEM((_C,), jnp.float32),
        pltpu.VMEM((_C, _CHUNK), jnp.float32),
        pltpu.VMEM((_CHUNK,), jnp.float32),
        pltpu.VMEM((_L,), jnp.float32),
        pltpu.SemaphoreType.DMA,
    ],
)
def _sc_loss(proj_hbm, hm_hbm, center_hbm, out_hbm, cen_v, ch_v, hm_v, acc_v, sem):
    wid = lax.axis_index("s") * 2 + lax.axis_index("c")

    pltpu.sync_copy(center_hbm, cen_v)
    c = cen_v[...]
    cs = [c[ch] for ch in range(_C)]
    ssc = cs[0] * cs[0]
    for ch in range(1, _C):
        ssc = ssc + cs[ch] * cs[ch]
    sv = jnp.full((_L,), ssc)
    r0 = _rsqrt16(sv)
    denom = sv * r0 + EPS
    cnb = [jnp.full((_L,), cs[ch]) / denom for ch in range(_C)]

    def inner(j, acc):
        base = j * _L
        ss = None
        dot = None
        for ch in range(_C):
            v = ch_v[ch, pl.ds(base, _L)]
            ss = v * v if ss is None else ss + v * v
            dot = cnb[ch] * v if dot is None else dot + cnb[ch] * v
        r = _rsqrt16(ss)
        sim = dot / (ss * r + EPS)
        d = sim - hm_v[pl.ds(base, _L)]
        return acc + d * d

    def chunk_body(k, acc):
        copies = [
            pltpu.make_async_copy(proj_hbm.at[ch, wid, k], ch_v.at[ch], sem)
            for ch in range(_C)
        ]
        copies.append(pltpu.make_async_copy(hm_hbm.at[wid, k], hm_v, sem))
        for cp in copies:
            cp.start()
        for cp in copies:
            cp.wait()
        return lax.fori_loop(0, _CHUNK // _L, inner, acc)

    acc = lax.fori_loop(0, _NCH, chunk_body, jnp.zeros((_L,), jnp.float32))

    acc_v[...] = acc
    pltpu.sync_copy(acc_v, out_hbm.at[wid])


def kernel(proj, hm, cluster_center, cluster_ind):
    center = jnp.take(cluster_center, cluster_ind, axis=0)
    center = jax.lax.stop_gradient(center)
    proj3 = proj.reshape(_C, _NW, _NCH, _CHUNK)
    hm3 = hm.reshape(_NW, _NCH, _CHUNK)

    out = _sc_loss(proj3, hm3, center)
    loss = jnp.sum(out) * (1.0 / _N)
    return (loss, loss * 0.0, loss)
